# baseline (speedup 1.0000x reference)
import functools

import jax
import jax.numpy as jnp
from jax import lax
from jax.experimental import pallas as pl
from jax.experimental.pallas import tpu as pltpu
from jax.experimental.pallas import tpu_sc as plsc

_ROWS = 16384
_HID = 2048
_EXPERTS = 64
_BR = 512
_NW = 32
_L = 16
_GROUPS = _BR // _L


def _logits_kernel(x_ref, w_ref, out_ref):
    out_ref[0] = jax.lax.dot_general(
        w_ref[...], x_ref[...], (((1,), (1,)), ((), ())),
        preferred_element_type=jnp.float32,
    )


def _tc_logits(x, w):
    return pl.pallas_call(
        _logits_kernel,
        grid=(_ROWS // _BR,),
        in_specs=[
            pl.BlockSpec((_BR, _HID), lambda i: (i, 0)),
            pl.BlockSpec((_EXPERTS, _HID), lambda i: (0, 0)),
        ],
        out_specs=pl.BlockSpec((1, _EXPERTS, _BR), lambda i: (i, 0, 0)),
        out_shape=jax.ShapeDtypeStruct((_NW, _EXPERTS, _BR), jnp.float32),
    )(x, w)


def _sc_topk(logits_blk):
    mesh = plsc.VectorSubcoreMesh(core_axis_name="c", subcore_axis_name="s")

    @functools.partial(
        pl.kernel,
        mesh=mesh,
        out_type=[
            jax.ShapeDtypeStruct((2 * _ROWS,), jnp.float32),
            jax.ShapeDtypeStruct((2 * _ROWS,), jnp.int32),
        ],
        scratch_types=[
            pltpu.VMEM((_EXPERTS, _BR), jnp.float32),
            pltpu.VMEM((2 * _BR,), jnp.float32),
            pltpu.VMEM((2 * _BR,), jnp.int32),
        ],
        compiler_params=pltpu.CompilerParams(needs_layout_passes=False),
    )
    def body(lg_hbm, val_hbm, idx_hbm, lg_v, val_v, idx_v):
        wid = lax.axis_index("s") * 2 + lax.axis_index("c")
        pltpu.sync_copy(lg_hbm.at[wid], lg_v)

        def group(g, _):
            neg = jnp.full((_L,), -jnp.inf, jnp.float32)
            zero = jnp.zeros((_L,), jnp.int32)
            m1, i1, m2, i2 = neg, zero, neg, zero
            for e in range(_EXPERTS):
                v = lg_v[e, pl.ds(g * _L, _L)]
                es = jnp.full((_L,), e, jnp.int32)
                gt1 = v > m1
                gt2 = v > m2
                m2 = jnp.where(gt1, m1, jnp.where(gt2, v, m2))
                i2 = jnp.where(gt1, i1, jnp.where(gt2, es, i2))
                m1 = jnp.where(gt1, v, m1)
                i1 = jnp.where(gt1, es, i1)
            w2 = jnp.exp(m2 - m1)
            inv = 1.0 / (1.0 + w2)
            pos = g * (2 * _L) + 2 * lax.iota(jnp.int32, _L)
            plsc.store_scatter(val_v, [pos], inv)
            plsc.store_scatter(val_v, [pos + 1], w2 * inv)
            plsc.store_scatter(idx_v, [pos], i1)
            plsc.store_scatter(idx_v, [pos + 1], i2)
            return ()

        lax.fori_loop(0, _GROUPS, group, (), unroll=False)
        base = wid * (2 * _BR)
        pltpu.sync_copy(val_v, val_hbm.at[pl.ds(base, 2 * _BR)])
        pltpu.sync_copy(idx_v, idx_hbm.at[pl.ds(base, 2 * _BR)])

    return body(logits_blk)


@jax.jit
def kernel(hidden_states, weight):
    lg = _tc_logits(hidden_states, weight)
    vals, idx = _sc_topk(lg)
    return (vals.reshape(_ROWS, 2), idx.reshape(_ROWS, 2))
